# Initial kernel scaffold; baseline (speedup 1.0000x reference)
import jax
import jax.numpy as jnp
from jax.experimental import pallas as pl


def kernel(x, edge_index, edge_attr, batch, Wn, bn, Wm, bm, Wo, bo, Wout, bout):
    raise NotImplementedError("write your pallas kernel here")


# speedup vs baseline: 8.9139x; 8.9139x over previous
import functools

import jax
import jax.numpy as jnp
from jax import lax
from jax.experimental import pallas as pl
from jax.experimental.pallas import tpu as pltpu
from jax.experimental.pallas import tpu_sc as plsc

N = 10000
E = 320000
D = 128
DE = 16
H = 128
G = 64
NUM_LAYERS = 3

NC = 2
NS = 16
PER_TILE = E // (NC * NS)
CH = 80
NCHUNK = PER_TILE // CH
ROWS_PER_TILE = N // NS
ZR = 125
FA = D + 16

_mesh = plsc.VectorSubcoreMesh(core_axis_name="c", subcore_axis_name="s")
_sc_params = pltpu.CompilerParams(use_tc_tiling_on_sc=False)


def _sc_spmv_body(src_hbm, dst_hbm, h_hbm, zeros_hbm, out_hbm,
                  src_v, dst_v, rows_v, stage_v, acc, sem):
    c = lax.axis_index("c")
    s = lax.axis_index("s")
    row0 = s * ROWS_PER_TILE
    pltpu.sync_copy(zeros_hbm, stage_v)
    for j in range(ROWS_PER_TILE // ZR):
        pltpu.sync_copy(stage_v, acc.at[pl.ds(row0 + j * ZR, ZR)])
    plsc.subcore_barrier()

    base = (c * NS + s) * PER_TILE

    def step(i, _):
        off = base + i * CH
        pltpu.sync_copy(src_hbm.at[pl.ds(off, CH)], src_v)
        pltpu.sync_copy(dst_hbm.at[pl.ds(off, CH)], dst_v)
        pltpu.async_copy(h_hbm.at[src_v], rows_v, sem).wait()
        pltpu.sync_copy(rows_v, acc.at[dst_v], add=True)
        return 0

    lax.fori_loop(0, NCHUNK, step, 0)
    plsc.subcore_barrier()

    out_base = c * N + row0
    for j in range(ROWS_PER_TILE // ZR):
        pltpu.sync_copy(acc.at[pl.ds(row0 + j * ZR, ZR)], stage_v)
        pltpu.sync_copy(stage_v, out_hbm.at[pl.ds(out_base + j * ZR, ZR)])


_sc_spmv = functools.partial(
    pl.kernel,
    out_type=jax.ShapeDtypeStruct((NC * N, H), jnp.float32),
    mesh=_mesh,
    compiler_params=_sc_params,
    scratch_types=[
        pltpu.VMEM((CH,), jnp.int32),
        pltpu.VMEM((CH,), jnp.int32),
        pltpu.VMEM((CH, H), jnp.float32),
        pltpu.VMEM((ZR, H), jnp.float32),
        pltpu.VMEM_SHARED((N, H), jnp.float32),
        pltpu.SemaphoreType.DMA,
    ],
)(_sc_spmv_body)


def _sc_attr_body(dst_hbm, attr_hbm, ones_hbm, zeros_hbm,
                  easum_hbm, deg_hbm,
                  dst_v, attr_v, ones_v, stage_v, acc_ea, acc_dg, sem):
    c = lax.axis_index("c")
    s = lax.axis_index("s")
    row0 = s * ROWS_PER_TILE
    pltpu.sync_copy(ones_hbm, ones_v)
    pltpu.sync_copy(zeros_hbm, stage_v)
    pltpu.sync_copy(stage_v, acc_ea.at[pl.ds(row0, ROWS_PER_TILE)])
    pltpu.sync_copy(stage_v, acc_dg.at[pl.ds(row0, ROWS_PER_TILE)])
    plsc.subcore_barrier()

    base = (c * NS + s) * PER_TILE

    def step(i, _):
        off = base + i * CH
        pltpu.sync_copy(dst_hbm.at[pl.ds(off, CH)], dst_v)
        pltpu.sync_copy(attr_hbm.at[pl.ds(off, CH)], attr_v)
        pltpu.sync_copy(attr_v, acc_ea.at[dst_v], add=True)
        pltpu.sync_copy(ones_v, acc_dg.at[dst_v], add=True)
        return 0

    lax.fori_loop(0, NCHUNK, step, 0)
    plsc.subcore_barrier()

    out_base = c * N + row0
    pltpu.sync_copy(acc_ea.at[pl.ds(row0, ROWS_PER_TILE)], stage_v)
    pltpu.sync_copy(stage_v, easum_hbm.at[pl.ds(out_base, ROWS_PER_TILE)])
    pltpu.sync_copy(acc_dg.at[pl.ds(row0, ROWS_PER_TILE)], stage_v)
    pltpu.sync_copy(stage_v, deg_hbm.at[pl.ds(out_base, ROWS_PER_TILE)])


_sc_attr = functools.partial(
    pl.kernel,
    out_type=(jax.ShapeDtypeStruct((NC * N, DE), jnp.float32),
              jax.ShapeDtypeStruct((NC * N, DE), jnp.float32)),
    mesh=_mesh,
    compiler_params=_sc_params,
    scratch_types=[
        pltpu.VMEM((CH,), jnp.int32),
        pltpu.VMEM((CH, DE), jnp.float32),
        pltpu.VMEM((CH, DE), jnp.float32),
        pltpu.VMEM((ROWS_PER_TILE, DE), jnp.float32),
        pltpu.VMEM_SHARED((N, DE), jnp.float32),
        pltpu.VMEM_SHARED((N, DE), jnp.float32),
        pltpu.SemaphoreType.DMA,
    ],
)(_sc_attr_body)


BN = 2000


def _tc_h0_body(x_ref, wn_ref, bn_ref, o_ref):
    o_ref[...] = jnp.maximum(
        jnp.dot(x_ref[...], wn_ref[...], preferred_element_type=jnp.float32,
                precision=lax.Precision.HIGHEST)
        + bn_ref[...], 0.0)


def _tc_prep_body(ax0_ref, ax1_ref, e0_ref, e1_ref, g0_ref, g1_ref,
                  x_ref, h0_ref, wm3_ref, bm_ref, c_ref, axf_ref, degf_ref):
    wm3 = wm3_ref[...]
    ea = e0_ref[...] + e1_ref[...]
    c_ref[...] = (jnp.dot(ea, wm3, preferred_element_type=jnp.float32,
                precision=lax.Precision.HIGHEST)
                  + jnp.sum(wm3, axis=0, keepdims=True)
                  + bm_ref[...] + h0_ref[...])
    axf_ref[...] = ax0_ref[...] + ax1_ref[...] + x_ref[...]
    degf_ref[...] = g0_ref[:, :1] + g1_ref[:, :1] + 1.0


def _tc_layer_body(h_ref, s0_ref, s1_ref, degf_ref, c_ref,
                   wm1_ref, wm2_ref, o_ref):
    h = h_ref[...]
    t = jnp.dot(degf_ref[...] * h, wm1_ref[...],
                preferred_element_type=jnp.float32,
                precision=lax.Precision.HIGHEST)
    t = t + jnp.dot(s0_ref[...] + s1_ref[...] + h, wm2_ref[...],
                    preferred_element_type=jnp.float32,
                precision=lax.Precision.HIGHEST)
    o_ref[...] = jnp.maximum(t + c_ref[...], 0.0)


def _tc_final_body(h_ref, s0_ref, s1_ref, degf_ref, axf_ref, batch_ref,
                   wo1_ref, wo2_ref, wo3_ref, bo_ref, wout_ref, o_ref):
    i = pl.program_id(0)
    h = h_ref[...]
    t = jnp.dot(degf_ref[...] * h, wo1_ref[...],
                preferred_element_type=jnp.float32,
                precision=lax.Precision.HIGHEST)
    t = t + jnp.dot(s0_ref[...] + s1_ref[...] + h, wo2_ref[...],
                    preferred_element_type=jnp.float32,
                precision=lax.Precision.HIGHEST)
    t = t + jnp.dot(axf_ref[...], wo3_ref[...],
                    preferred_element_type=jnp.float32,
                precision=lax.Precision.HIGHEST)
    out = jnp.maximum(t + bo_ref[...], 0.0)
    z = jnp.dot(out, wout_ref[...], preferred_element_type=jnp.float32,
                precision=lax.Precision.HIGHEST)
    onehot = (batch_ref[...] ==
              lax.broadcasted_iota(jnp.int32, (1, G), 1)).astype(jnp.float32)
    pooled = lax.dot_general(onehot, z, (((0,), (0,)), ((), ())),
                             preferred_element_type=jnp.float32,
                precision=lax.Precision.HIGHEST)

    @pl.when(i == 0)
    def _():
        o_ref[...] = jnp.zeros((G, 1), jnp.float32)

    o_ref[...] += pooled


def _row_spec(bf):
    return pl.BlockSpec((BN, bf), lambda i: (i, 0))


def _full_spec(shape):
    nd = len(shape)
    return pl.BlockSpec(shape, lambda i: (0,) * nd)


def _part_specs(bf):
    return (pl.BlockSpec((BN, bf), lambda i: (i, 0)),
            pl.BlockSpec((BN, bf), lambda i: (i + N // BN, 0)))


def kernel(x, edge_index, edge_attr, batch, Wn, bn, Wm, bm, Wo, bo, Wout, bout):
    f32 = jnp.float32
    src = edge_index[0].astype(jnp.int32)
    dst = edge_index[1].astype(jnp.int32)
    ones_a = jnp.ones((CH, DE), f32)
    ze = jnp.zeros((ROWS_PER_TILE, DE), f32)
    zh = jnp.zeros((ZR, H), f32)

    h0 = pl.pallas_call(
        _tc_h0_body,
        grid=(N // BN,),
        in_specs=[_row_spec(D), _full_spec((D, H)), _full_spec((1, H))],
        out_specs=_row_spec(H),
        out_shape=jax.ShapeDtypeStruct((N, H), f32),
    )(x, Wn, bn.reshape(1, H))

    ax = _sc_spmv(src, dst, x, zh)
    easum, degp = _sc_attr(dst, edge_attr, ones_a, ze)

    p0, p1 = _part_specs(D)
    q0, q1 = _part_specs(DE)
    g0, g1 = _part_specs(DE)
    C, Axf, degf = pl.pallas_call(
        _tc_prep_body,
        grid=(N // BN,),
        in_specs=[p0, p1, q0, q1, g0, g1, _row_spec(D), _row_spec(H),
                  _full_spec((DE, H)), _full_spec((1, H))],
        out_specs=(_row_spec(H), _row_spec(D), _row_spec(1)),
        out_shape=(jax.ShapeDtypeStruct((N, H), f32),
                   jax.ShapeDtypeStruct((N, D), f32),
                   jax.ShapeDtypeStruct((N, 1), f32)),
    )(ax, ax, easum, easum, degp, degp, x, h0, Wm[2 * H:], bm.reshape(1, H))

    r0, r1 = _part_specs(H)
    h = h0
    for _ in range(NUM_LAYERS):
        sp = _sc_spmv(src, dst, h, zh)
        h = pl.pallas_call(
            _tc_layer_body,
            grid=(N // BN,),
            in_specs=[_row_spec(H), r0, r1, _row_spec(1), _row_spec(H),
                      _full_spec((H, H)), _full_spec((H, H))],
            out_specs=_row_spec(H),
            out_shape=jax.ShapeDtypeStruct((N, H), f32),
        )(h, sp, sp, degf, C, Wm[:H], Wm[H:2 * H])

    sp = _sc_spmv(src, dst, h, zh)
    pooled = pl.pallas_call(
        _tc_final_body,
        grid=(N // BN,),
        in_specs=[_row_spec(H), r0, r1, _row_spec(1), _row_spec(D),
                  _row_spec(1),
                  _full_spec((H, H)), _full_spec((H, H)), _full_spec((D, H)),
                  _full_spec((1, H)), _full_spec((H, 1))],
        out_specs=_full_spec((G, 1)),
        out_shape=jax.ShapeDtypeStruct((G, 1), f32),
    )(h, sp, sp, degf, Axf, batch.reshape(N, 1).astype(jnp.int32),
      Wo[:H], Wo[H:2 * H], Wo[2 * H:], bo.reshape(1, H), Wout)

    return pooled + bout


# speedup vs baseline: 17.3839x; 1.9502x over previous
import functools

import jax
import jax.numpy as jnp
from jax import lax
from jax.experimental import pallas as pl
from jax.experimental.pallas import tpu as pltpu
from jax.experimental.pallas import tpu_sc as plsc

N = 10000
E = 320000
D = 128
DE = 16
H = 128
G = 64
NUM_LAYERS = 3

NC = 2
NS = 16
PER_TILE = E // (NC * NS)
CH = 80
NCHUNK = PER_TILE // CH
ROWS_PER_TILE = N // NS
ZR = 125
_WB_CHUNKS = [(i * CH, CH) for i in range(ROWS_PER_TILE // CH)] + [
    ((ROWS_PER_TILE // CH) * CH, ROWS_PER_TILE % CH)]
FA = D + 16

_mesh = plsc.VectorSubcoreMesh(core_axis_name="c", subcore_axis_name="s")
_sc_params = pltpu.CompilerParams(use_tc_tiling_on_sc=False)


def _sc_spmv_body(src_hbm, dst2_hbm, h_hbm, zeros_hbm, out_hbm,
                  srcall_v, dstall_v, rows0_v, rows1_v, acc,
                  sem0, sem1):
    c = lax.axis_index("c")
    s = lax.axis_index("s")
    wid = c * NS + s
    row0 = s * ROWS_PER_TILE
    pltpu.sync_copy(src_hbm.at[pl.ds(wid * PER_TILE, PER_TILE)], srcall_v)
    pltpu.sync_copy(dst2_hbm.at[pl.ds(wid * NCHUNK, NCHUNK)], dstall_v)
    pltpu.sync_copy(zeros_hbm, rows0_v)
    for r0, rn in _WB_CHUNKS:
        pltpu.sync_copy(rows0_v.at[pl.ds(0, rn)], acc.at[pl.ds(row0 + r0, rn)])
    plsc.subcore_barrier()

    bufs = ((rows0_v, sem0), (rows1_v, sem1))

    def start_gather(ch, rows, sem):
        pltpu.async_copy(h_hbm.at[srcall_v.at[pl.ds(ch * CH, CH)]], rows, sem)

    def wait_gather(rows, sem):
        pltpu.make_async_copy(h_hbm.at[pl.ds(0, CH)], rows, sem).wait()

    def scatter(ch, rows):
        pltpu.sync_copy(rows, acc.at[dstall_v.at[ch]], add=True)

    start_gather(0, rows0_v, sem0)
    start_gather(1, rows1_v, sem1)

    def superstep(t, _):
        for b, (rows, sem) in enumerate(bufs):
            ch = 2 * t + b
            wait_gather(rows, sem)
            scatter(ch, rows)

            @pl.when(ch + 2 < NCHUNK)
            def _():
                start_gather(ch + 2, rows, sem)
        return 0

    lax.fori_loop(0, NCHUNK // 2, superstep, 0)
    wait_gather(rows0_v, sem0)
    scatter(NCHUNK - 1, rows0_v)
    plsc.subcore_barrier()

    out_base = c * N + row0
    for r0, rn in _WB_CHUNKS:
        pltpu.sync_copy(acc.at[pl.ds(row0 + r0, rn)], rows0_v.at[pl.ds(0, rn)])
        pltpu.sync_copy(rows0_v.at[pl.ds(0, rn)],
                        out_hbm.at[pl.ds(out_base + r0, rn)])


_sc_spmv = functools.partial(
    pl.kernel,
    out_type=jax.ShapeDtypeStruct((NC * N, H), jnp.float32),
    mesh=_mesh,
    compiler_params=_sc_params,
    scratch_types=[
        pltpu.VMEM((PER_TILE,), jnp.int32),
        pltpu.VMEM((NCHUNK, CH), jnp.int32),
        pltpu.VMEM((CH, H), jnp.float32),
        pltpu.VMEM((CH, H), jnp.float32),
        pltpu.VMEM_SHARED((N, H), jnp.float32),
        pltpu.SemaphoreType.DMA,
        pltpu.SemaphoreType.DMA,
    ],
)(_sc_spmv_body)


def _sc_attr_body(dst_hbm, attr_hbm, ones_hbm, zeros_hbm,
                  easum_hbm, deg_hbm,
                  dst_v, attr_v, ones_v, stage_v, acc_ea, acc_dg, sem):
    c = lax.axis_index("c")
    s = lax.axis_index("s")
    row0 = s * ROWS_PER_TILE
    pltpu.sync_copy(ones_hbm, ones_v)
    pltpu.sync_copy(zeros_hbm, stage_v)
    pltpu.sync_copy(stage_v, acc_ea.at[pl.ds(row0, ROWS_PER_TILE)])
    pltpu.sync_copy(stage_v, acc_dg.at[pl.ds(row0, ROWS_PER_TILE)])
    plsc.subcore_barrier()

    base = (c * NS + s) * PER_TILE

    def step(i, _):
        off = base + i * CH
        pltpu.sync_copy(dst_hbm.at[pl.ds(off, CH)], dst_v)
        pltpu.sync_copy(attr_hbm.at[pl.ds(off, CH)], attr_v)
        pltpu.sync_copy(attr_v, acc_ea.at[dst_v], add=True)
        pltpu.sync_copy(ones_v, acc_dg.at[dst_v], add=True)
        return 0

    lax.fori_loop(0, NCHUNK, step, 0)
    plsc.subcore_barrier()

    out_base = c * N + row0
    pltpu.sync_copy(acc_ea.at[pl.ds(row0, ROWS_PER_TILE)], stage_v)
    pltpu.sync_copy(stage_v, easum_hbm.at[pl.ds(out_base, ROWS_PER_TILE)])
    pltpu.sync_copy(acc_dg.at[pl.ds(row0, ROWS_PER_TILE)], stage_v)
    pltpu.sync_copy(stage_v, deg_hbm.at[pl.ds(out_base, ROWS_PER_TILE)])


_sc_attr = functools.partial(
    pl.kernel,
    out_type=(jax.ShapeDtypeStruct((NC * N, DE), jnp.float32),
              jax.ShapeDtypeStruct((NC * N, DE), jnp.float32)),
    mesh=_mesh,
    compiler_params=_sc_params,
    scratch_types=[
        pltpu.VMEM((CH,), jnp.int32),
        pltpu.VMEM((CH, DE), jnp.float32),
        pltpu.VMEM((CH, DE), jnp.float32),
        pltpu.VMEM((ROWS_PER_TILE, DE), jnp.float32),
        pltpu.VMEM_SHARED((N, DE), jnp.float32),
        pltpu.VMEM_SHARED((N, DE), jnp.float32),
        pltpu.SemaphoreType.DMA,
    ],
)(_sc_attr_body)


BN = 2000


def _tc_h0_body(x_ref, wn_ref, bn_ref, o_ref):
    o_ref[...] = jnp.maximum(
        jnp.dot(x_ref[...], wn_ref[...], preferred_element_type=jnp.float32,
                precision=lax.Precision.HIGHEST)
        + bn_ref[...], 0.0)


def _tc_prep_body(ax0_ref, ax1_ref, e0_ref, e1_ref, g0_ref, g1_ref,
                  x_ref, h0_ref, wm3_ref, bm_ref, c_ref, axf_ref, degf_ref):
    wm3 = wm3_ref[...]
    ea = e0_ref[...] + e1_ref[...]
    c_ref[...] = (jnp.dot(ea, wm3, preferred_element_type=jnp.float32,
                precision=lax.Precision.HIGHEST)
                  + jnp.sum(wm3, axis=0, keepdims=True)
                  + bm_ref[...] + h0_ref[...])
    axf_ref[...] = ax0_ref[...] + ax1_ref[...] + x_ref[...]
    degf_ref[...] = g0_ref[:, :1] + g1_ref[:, :1] + 1.0


def _tc_layer_body(h_ref, s0_ref, s1_ref, degf_ref, c_ref,
                   wm1_ref, wm2_ref, o_ref):
    h = h_ref[...]
    t = jnp.dot(degf_ref[...] * h, wm1_ref[...],
                preferred_element_type=jnp.float32,
                precision=lax.Precision.HIGHEST)
    t = t + jnp.dot(s0_ref[...] + s1_ref[...] + h, wm2_ref[...],
                    preferred_element_type=jnp.float32,
                precision=lax.Precision.HIGHEST)
    o_ref[...] = jnp.maximum(t + c_ref[...], 0.0)


def _tc_final_body(h_ref, s0_ref, s1_ref, degf_ref, axf_ref, batch_ref,
                   wo1_ref, wo2_ref, wo3_ref, bo_ref, wout_ref, o_ref):
    i = pl.program_id(0)
    h = h_ref[...]
    t = jnp.dot(degf_ref[...] * h, wo1_ref[...],
                preferred_element_type=jnp.float32,
                precision=lax.Precision.HIGHEST)
    t = t + jnp.dot(s0_ref[...] + s1_ref[...] + h, wo2_ref[...],
                    preferred_element_type=jnp.float32,
                precision=lax.Precision.HIGHEST)
    t = t + jnp.dot(axf_ref[...], wo3_ref[...],
                    preferred_element_type=jnp.float32,
                precision=lax.Precision.HIGHEST)
    out = jnp.maximum(t + bo_ref[...], 0.0)
    z = jnp.dot(out, wout_ref[...], preferred_element_type=jnp.float32,
                precision=lax.Precision.HIGHEST)
    onehot = (batch_ref[...] ==
              lax.broadcasted_iota(jnp.int32, (1, G), 1)).astype(jnp.float32)
    pooled = lax.dot_general(onehot, z, (((0,), (0,)), ((), ())),
                             preferred_element_type=jnp.float32,
                precision=lax.Precision.HIGHEST)

    @pl.when(i == 0)
    def _():
        o_ref[...] = jnp.zeros((G, 1), jnp.float32)

    o_ref[...] += pooled


def _row_spec(bf):
    return pl.BlockSpec((BN, bf), lambda i: (i, 0))


def _full_spec(shape):
    nd = len(shape)
    return pl.BlockSpec(shape, lambda i: (0,) * nd)


def _part_specs(bf):
    return (pl.BlockSpec((BN, bf), lambda i: (i, 0)),
            pl.BlockSpec((BN, bf), lambda i: (i + N // BN, 0)))


def kernel(x, edge_index, edge_attr, batch, Wn, bn, Wm, bm, Wo, bo, Wout, bout):
    f32 = jnp.float32
    src = edge_index[0].astype(jnp.int32)
    dst = edge_index[1].astype(jnp.int32)
    dst2 = dst.reshape(E // CH, CH)
    ones_a = jnp.ones((CH, DE), f32)
    ze = jnp.zeros((ROWS_PER_TILE, DE), f32)
    zh = jnp.zeros((CH, H), f32)

    h0 = pl.pallas_call(
        _tc_h0_body,
        grid=(N // BN,),
        in_specs=[_row_spec(D), _full_spec((D, H)), _full_spec((1, H))],
        out_specs=_row_spec(H),
        out_shape=jax.ShapeDtypeStruct((N, H), f32),
    )(x, Wn, bn.reshape(1, H))

    ax = _sc_spmv(src, dst2, x, zh)
    easum, degp = _sc_attr(dst, edge_attr, ones_a, ze)

    p0, p1 = _part_specs(D)
    q0, q1 = _part_specs(DE)
    g0, g1 = _part_specs(DE)
    C, Axf, degf = pl.pallas_call(
        _tc_prep_body,
        grid=(N // BN,),
        in_specs=[p0, p1, q0, q1, g0, g1, _row_spec(D), _row_spec(H),
                  _full_spec((DE, H)), _full_spec((1, H))],
        out_specs=(_row_spec(H), _row_spec(D), _row_spec(1)),
        out_shape=(jax.ShapeDtypeStruct((N, H), f32),
                   jax.ShapeDtypeStruct((N, D), f32),
                   jax.ShapeDtypeStruct((N, 1), f32)),
    )(ax, ax, easum, easum, degp, degp, x, h0, Wm[2 * H:], bm.reshape(1, H))

    r0, r1 = _part_specs(H)
    h = h0
    for _ in range(NUM_LAYERS):
        sp = _sc_spmv(src, dst2, h, zh)
        h = pl.pallas_call(
            _tc_layer_body,
            grid=(N // BN,),
            in_specs=[_row_spec(H), r0, r1, _row_spec(1), _row_spec(H),
                      _full_spec((H, H)), _full_spec((H, H))],
            out_specs=_row_spec(H),
            out_shape=jax.ShapeDtypeStruct((N, H), f32),
        )(h, sp, sp, degf, C, Wm[:H], Wm[H:2 * H])

    sp = _sc_spmv(src, dst2, h, zh)
    pooled = pl.pallas_call(
        _tc_final_body,
        grid=(N // BN,),
        in_specs=[_row_spec(H), r0, r1, _row_spec(1), _row_spec(D),
                  _row_spec(1),
                  _full_spec((H, H)), _full_spec((H, H)), _full_spec((D, H)),
                  _full_spec((1, H)), _full_spec((H, 1))],
        out_specs=_full_spec((G, 1)),
        out_shape=jax.ShapeDtypeStruct((G, 1), f32),
    )(h, sp, sp, degf, Axf, batch.reshape(N, 1).astype(jnp.int32),
      Wo[:H], Wo[H:2 * H], Wo[2 * H:], bo.reshape(1, H), Wout)

    return pooled + bout


# speedup vs baseline: 22.0323x; 1.2674x over previous
import functools

import jax
import jax.numpy as jnp
from jax import lax
from jax.experimental import pallas as pl
from jax.experimental.pallas import tpu as pltpu
from jax.experimental.pallas import tpu_sc as plsc

N = 10000
E = 320000
D = 128
DE = 16
H = 128
G = 64
NUM_LAYERS = 3

NC = 2
NS = 16
PER_TILE = E // (NC * NS)
CH = 80
NCHUNK = PER_TILE // CH
ROWS_PER_TILE = N // NS
ZR = 125
_WB_CHUNKS = [(i * CH, CH) for i in range(ROWS_PER_TILE // CH)] + [
    ((ROWS_PER_TILE // CH) * CH, ROWS_PER_TILE % CH)]
FA = D + 16

_mesh = plsc.VectorSubcoreMesh(core_axis_name="c", subcore_axis_name="s")
_sc_params = pltpu.CompilerParams(use_tc_tiling_on_sc=False)


def _sc_spmv_body(src_hbm, dst2_hbm, h_hbm, zeros_hbm, out_hbm,
                  srcall_v, dstall_v, rows0_v, rows1_v, rows2_v, acc,
                  sem0, sem1, sem2):
    c = lax.axis_index("c")
    s = lax.axis_index("s")
    wid = c * NS + s
    row0 = s * ROWS_PER_TILE
    pltpu.sync_copy(src_hbm.at[pl.ds(wid * PER_TILE, PER_TILE)], srcall_v)
    pltpu.sync_copy(dst2_hbm.at[pl.ds(wid * NCHUNK, NCHUNK)], dstall_v)
    pltpu.sync_copy(zeros_hbm, rows0_v)
    for r0, rn in _WB_CHUNKS:
        pltpu.sync_copy(rows0_v.at[pl.ds(0, rn)], acc.at[pl.ds(row0 + r0, rn)])
    plsc.subcore_barrier()

    bufs = ((rows0_v, sem0), (rows1_v, sem1), (rows2_v, sem2))
    nbuf = len(bufs)

    def start_gather(ch, rows, sem):
        pltpu.async_copy(h_hbm.at[srcall_v.at[pl.ds(ch * CH, CH)]], rows, sem)

    def wait_gather(rows, sem):
        pltpu.make_async_copy(h_hbm.at[pl.ds(0, CH)], rows, sem).wait()

    def scatter(ch, rows):
        pltpu.sync_copy(rows, acc.at[dstall_v.at[ch]], add=True)

    for b, (rows, sem) in enumerate(bufs):
        start_gather(b, rows, sem)

    nfull = NCHUNK // nbuf

    def superstep(t, _):
        for b, (rows, sem) in enumerate(bufs):
            ch = nbuf * t + b
            wait_gather(rows, sem)
            scatter(ch, rows)

            @pl.when(ch + nbuf < NCHUNK)
            def _():
                start_gather(ch + nbuf, rows, sem)
        return 0

    lax.fori_loop(0, nfull, superstep, 0)
    for b in range(NCHUNK - nfull * nbuf):
        rows, sem = bufs[b]
        wait_gather(rows, sem)
        scatter(nfull * nbuf + b, rows)
    plsc.subcore_barrier()

    out_base = c * N + row0
    for r0, rn in _WB_CHUNKS:
        pltpu.sync_copy(acc.at[pl.ds(row0 + r0, rn)], rows0_v.at[pl.ds(0, rn)])
        pltpu.sync_copy(rows0_v.at[pl.ds(0, rn)],
                        out_hbm.at[pl.ds(out_base + r0, rn)])


_sc_spmv = functools.partial(
    pl.kernel,
    out_type=jax.ShapeDtypeStruct((NC * N, H), jnp.float32),
    mesh=_mesh,
    compiler_params=_sc_params,
    scratch_types=[
        pltpu.VMEM((PER_TILE,), jnp.int32),
        pltpu.VMEM((NCHUNK, CH), jnp.int32),
        pltpu.VMEM((CH, H), jnp.float32),
        pltpu.VMEM((CH, H), jnp.float32),
        pltpu.VMEM((CH, H), jnp.float32),
        pltpu.VMEM_SHARED((N, H), jnp.float32),
        pltpu.SemaphoreType.DMA,
        pltpu.SemaphoreType.DMA,
        pltpu.SemaphoreType.DMA,
    ],
)(_sc_spmv_body)


def _sc_attr_body(dst2_hbm, attr2_hbm, ones_hbm, zeros_hbm,
                  easum_hbm, deg_hbm,
                  dstall_v, at0_v, at1_v, ones_v, stage_v,
                  acc_ea, acc_dg, sem0, sem1):
    c = lax.axis_index("c")
    s = lax.axis_index("s")
    wid = c * NS + s
    row0 = s * ROWS_PER_TILE
    pltpu.sync_copy(dst2_hbm.at[pl.ds(wid * NCHUNK, NCHUNK)], dstall_v)
    pltpu.sync_copy(ones_hbm, ones_v)
    pltpu.sync_copy(zeros_hbm, stage_v)
    pltpu.sync_copy(stage_v, acc_ea.at[pl.ds(row0, ROWS_PER_TILE)])
    pltpu.sync_copy(stage_v, acc_dg.at[pl.ds(row0, ROWS_PER_TILE)])
    plsc.subcore_barrier()

    base = wid * PER_TILE
    bufs = ((at0_v, sem0), (at1_v, sem1))
    nbuf = len(bufs)

    def start_load(ch, buf, sem):
        pltpu.async_copy(attr2_hbm.at[pl.ds(base + ch * CH, CH)], buf, sem)

    def wait_load(buf, sem):
        pltpu.make_async_copy(attr2_hbm.at[pl.ds(0, CH)], buf, sem).wait()

    for b, (buf, sem) in enumerate(bufs):
        start_load(b, buf, sem)

    nfull = NCHUNK // nbuf

    def step(t, _):
        for b, (buf, sem) in enumerate(bufs):
            ch = nbuf * t + b
            wait_load(buf, sem)
            pltpu.sync_copy(buf, acc_ea.at[dstall_v.at[ch]], add=True)
            pltpu.sync_copy(ones_v, acc_dg.at[dstall_v.at[ch]], add=True)

            @pl.when(ch + nbuf < NCHUNK)
            def _():
                start_load(ch + nbuf, buf, sem)
        return 0

    lax.fori_loop(0, nfull, step, 0)
    for b in range(NCHUNK - nfull * nbuf):
        buf, sem = bufs[b]
        ch = nfull * nbuf + b
        wait_load(buf, sem)
        pltpu.sync_copy(buf, acc_ea.at[dstall_v.at[ch]], add=True)
        pltpu.sync_copy(ones_v, acc_dg.at[dstall_v.at[ch]], add=True)
    plsc.subcore_barrier()

    out_base = c * N + row0
    pltpu.sync_copy(acc_ea.at[pl.ds(row0, ROWS_PER_TILE)], stage_v)
    pltpu.sync_copy(stage_v, easum_hbm.at[pl.ds(out_base, ROWS_PER_TILE)])
    pltpu.sync_copy(acc_dg.at[pl.ds(row0, ROWS_PER_TILE)], stage_v)
    pltpu.sync_copy(stage_v, deg_hbm.at[pl.ds(out_base, ROWS_PER_TILE)])


_sc_attr = functools.partial(
    pl.kernel,
    out_type=(jax.ShapeDtypeStruct((NC * N, DE), jnp.float32),
              jax.ShapeDtypeStruct((NC * N, DE), jnp.float32)),
    mesh=_mesh,
    compiler_params=_sc_params,
    scratch_types=[
        pltpu.VMEM((NCHUNK, CH), jnp.int32),
        pltpu.VMEM((CH, DE), jnp.float32),
        pltpu.VMEM((CH, DE), jnp.float32),
        pltpu.VMEM((CH, DE), jnp.float32),
        pltpu.VMEM((ROWS_PER_TILE, DE), jnp.float32),
        pltpu.VMEM_SHARED((N, DE), jnp.float32),
        pltpu.VMEM_SHARED((N, DE), jnp.float32),
        pltpu.SemaphoreType.DMA,
        pltpu.SemaphoreType.DMA,
    ],
)(_sc_attr_body)


BN = 2000


def _tc_h0_body(x_ref, wn_ref, bn_ref, o_ref):
    o_ref[...] = jnp.maximum(
        jnp.dot(x_ref[...], wn_ref[...], preferred_element_type=jnp.float32,
                precision=lax.Precision.HIGHEST)
        + bn_ref[...], 0.0)


def _tc_prep_body(ax0_ref, ax1_ref, e0_ref, e1_ref, g0_ref, g1_ref,
                  x_ref, h0_ref, wm3_ref, bm_ref, c_ref, axf_ref, degf_ref):
    wm3 = wm3_ref[...]
    ea = e0_ref[...] + e1_ref[...]
    c_ref[...] = (jnp.dot(ea, wm3, preferred_element_type=jnp.float32,
                precision=lax.Precision.HIGHEST)
                  + jnp.sum(wm3, axis=0, keepdims=True)
                  + bm_ref[...] + h0_ref[...])
    axf_ref[...] = ax0_ref[...] + ax1_ref[...] + x_ref[...]
    degf_ref[...] = g0_ref[:, :1] + g1_ref[:, :1] + 1.0


def _tc_layer_body(h_ref, s0_ref, s1_ref, degf_ref, c_ref,
                   wm1_ref, wm2_ref, o_ref):
    h = h_ref[...]
    t = jnp.dot(degf_ref[...] * h, wm1_ref[...],
                preferred_element_type=jnp.float32,
                precision=lax.Precision.HIGHEST)
    t = t + jnp.dot(s0_ref[...] + s1_ref[...] + h, wm2_ref[...],
                    preferred_element_type=jnp.float32,
                precision=lax.Precision.HIGHEST)
    o_ref[...] = jnp.maximum(t + c_ref[...], 0.0)


def _tc_final_body(h_ref, s0_ref, s1_ref, degf_ref, axf_ref, batch_ref,
                   wo1_ref, wo2_ref, wo3_ref, bo_ref, wout_ref, o_ref):
    i = pl.program_id(0)
    h = h_ref[...]
    t = jnp.dot(degf_ref[...] * h, wo1_ref[...],
                preferred_element_type=jnp.float32,
                precision=lax.Precision.HIGHEST)
    t = t + jnp.dot(s0_ref[...] + s1_ref[...] + h, wo2_ref[...],
                    preferred_element_type=jnp.float32,
                precision=lax.Precision.HIGHEST)
    t = t + jnp.dot(axf_ref[...], wo3_ref[...],
                    preferred_element_type=jnp.float32,
                precision=lax.Precision.HIGHEST)
    out = jnp.maximum(t + bo_ref[...], 0.0)
    z = jnp.dot(out, wout_ref[...], preferred_element_type=jnp.float32,
                precision=lax.Precision.HIGHEST)
    onehot = (batch_ref[...] ==
              lax.broadcasted_iota(jnp.int32, (1, G), 1)).astype(jnp.float32)
    pooled = lax.dot_general(onehot, z, (((0,), (0,)), ((), ())),
                             preferred_element_type=jnp.float32,
                precision=lax.Precision.HIGHEST)

    @pl.when(i == 0)
    def _():
        o_ref[...] = jnp.zeros((G, 1), jnp.float32)

    o_ref[...] += pooled


def _row_spec(bf):
    return pl.BlockSpec((BN, bf), lambda i: (i, 0))


def _full_spec(shape):
    nd = len(shape)
    return pl.BlockSpec(shape, lambda i: (0,) * nd)


def _part_specs(bf):
    return (pl.BlockSpec((BN, bf), lambda i: (i, 0)),
            pl.BlockSpec((BN, bf), lambda i: (i + N // BN, 0)))


def kernel(x, edge_index, edge_attr, batch, Wn, bn, Wm, bm, Wo, bo, Wout, bout):
    f32 = jnp.float32
    src = edge_index[0].astype(jnp.int32)
    dst = edge_index[1].astype(jnp.int32)
    dst2 = dst.reshape(E // CH, CH)
    ones_a = jnp.ones((CH, DE), f32)
    ze = jnp.zeros((ROWS_PER_TILE, DE), f32)
    zh = jnp.zeros((CH, H), f32)

    h0 = pl.pallas_call(
        _tc_h0_body,
        grid=(N // BN,),
        in_specs=[_row_spec(D), _full_spec((D, H)), _full_spec((1, H))],
        out_specs=_row_spec(H),
        out_shape=jax.ShapeDtypeStruct((N, H), f32),
    )(x, Wn, bn.reshape(1, H))

    ax = _sc_spmv(src, dst2, x, zh)
    easum, degp = _sc_attr(dst2, edge_attr, ones_a, ze)

    p0, p1 = _part_specs(D)
    q0, q1 = _part_specs(DE)
    g0, g1 = _part_specs(DE)
    C, Axf, degf = pl.pallas_call(
        _tc_prep_body,
        grid=(N // BN,),
        in_specs=[p0, p1, q0, q1, g0, g1, _row_spec(D), _row_spec(H),
                  _full_spec((DE, H)), _full_spec((1, H))],
        out_specs=(_row_spec(H), _row_spec(D), _row_spec(1)),
        out_shape=(jax.ShapeDtypeStruct((N, H), f32),
                   jax.ShapeDtypeStruct((N, D), f32),
                   jax.ShapeDtypeStruct((N, 1), f32)),
    )(ax, ax, easum, easum, degp, degp, x, h0, Wm[2 * H:], bm.reshape(1, H))

    r0, r1 = _part_specs(H)
    h = h0
    for _ in range(NUM_LAYERS):
        sp = _sc_spmv(src, dst2, h, zh)
        h = pl.pallas_call(
            _tc_layer_body,
            grid=(N // BN,),
            in_specs=[_row_spec(H), r0, r1, _row_spec(1), _row_spec(H),
                      _full_spec((H, H)), _full_spec((H, H))],
            out_specs=_row_spec(H),
            out_shape=jax.ShapeDtypeStruct((N, H), f32),
        )(h, sp, sp, degf, C, Wm[:H], Wm[H:2 * H])

    sp = _sc_spmv(src, dst2, h, zh)
    pooled = pl.pallas_call(
        _tc_final_body,
        grid=(N // BN,),
        in_specs=[_row_spec(H), r0, r1, _row_spec(1), _row_spec(D),
                  _row_spec(1),
                  _full_spec((H, H)), _full_spec((H, H)), _full_spec((D, H)),
                  _full_spec((1, H)), _full_spec((H, 1))],
        out_specs=_full_spec((G, 1)),
        out_shape=jax.ShapeDtypeStruct((G, 1), f32),
    )(h, sp, sp, degf, Axf, batch.reshape(N, 1).astype(jnp.int32),
      Wo[:H], Wo[H:2 * H], Wo[2 * H:], bo.reshape(1, H), Wout)

    return pooled + bout
